# Initial kernel scaffold; baseline (speedup 1.0000x reference)
import jax
import jax.numpy as jnp
from jax.experimental import pallas as pl


def kernel(x):
    raise NotImplementedError("write your pallas kernel here")


# speedup vs baseline: 15.9119x; 15.9119x over previous
import functools

import numpy as np
import jax
import jax.numpy as jnp
from jax import lax
from jax.experimental import pallas as pl
from jax.experimental.pallas import tpu as pltpu
from jax.experimental.pallas import tpu_sc as plsc

NROWS, CH, HH, WW = 8, 768, 24, 24
L = CH * HH * WW
MASK_RATIO = 0.3
LEN_KEEP = int(L * (1 - MASK_RATIO))
LANE = 128
SUB = L // LANE
ROW_CHUNKS = 8
BLK = SUB // ROW_CHUNKS
BLK_ELEMS = BLK * LANE

QUARTER = L // 4
NCHUNK = 8
CHUNK = QUARTER // NCHUNK
NVEC = CHUNK // 16
BIN_SHIFT = 10
HBINS = 1 << (23 - BIN_SHIFT)
LOW_MASK = (1 << BIN_SHIFT) - 1
POS_BITS = 19
QCAP = 64
PAD = 0x7FFFFFFF

_ROT_A = (13, 15, 26, 6)
_ROT_B = (17, 29, 16, 24)


def _np_threefry2x32(k0, k1, x0, x1):
    def rotl(x, r):
        return ((x << np.uint32(r)) | (x >> np.uint32(32 - r))).astype(np.uint32)

    k0, k1 = np.uint32(k0), np.uint32(k1)
    k2 = np.uint32(k0 ^ k1 ^ np.uint32(0x1BD11BDA))
    ks = (k0, k1, k2)
    x0 = (x0 + k0).astype(np.uint32)
    x1 = (x1 + k1).astype(np.uint32)
    for g in range(5):
        for r in (_ROT_A if g % 2 == 0 else _ROT_B):
            x0 = (x0 + x1).astype(np.uint32)
            x1 = rotl(x1, r)
            x1 = (x1 ^ x0).astype(np.uint32)
        x0 = (x0 + ks[(g + 1) % 3]).astype(np.uint32)
        x1 = (x1 + ks[(g + 2) % 3] + np.uint32(g + 1)).astype(np.uint32)
    return x0, x1


_FK0, _FK1 = _np_threefry2x32(
    np.uint32(0), np.uint32(42), np.array([0], np.uint32), np.array([1], np.uint32)
)
K0 = int(_FK0[0])
K1 = int(_FK1[0])
K2 = int(np.uint32(K0) ^ np.uint32(K1) ^ np.uint32(0x1BD11BDA))
_KS = (K0, K1, K2)

_ch_idx = np.asarray(np.random.default_rng(0).choice(CH, size=int(CH * 0.2), replace=False))
_chm = np.ones((CH,), np.float32)
_chm[_ch_idx] = 0.0
_CHM = np.repeat(_chm, HH * WW).reshape(SUB, LANE)


def _gen_body(u_ref):
    r = pl.program_id(0)
    k = pl.program_id(1)
    base = r * L + k * BLK_ELEMS
    i0 = lax.broadcasted_iota(jnp.int32, (BLK, LANE), 0)
    i1 = lax.broadcasted_iota(jnp.int32, (BLK, LANE), 1)
    cnt = (base + i0 * LANE + i1).astype(jnp.uint32)
    x0 = jnp.full((BLK, LANE), np.uint32(K0), jnp.uint32)
    x1 = cnt + np.uint32(K1)
    for g in range(5):
        for rot in (_ROT_A if g % 2 == 0 else _ROT_B):
            x0 = x0 + x1
            x1 = (x1 << np.uint32(rot)) | (x1 >> np.uint32(32 - rot))
            x1 = x1 ^ x0
        x0 = x0 + np.uint32(_KS[(g + 1) % 3])
        x1 = x1 + np.uint32((_KS[(g + 2) % 3] + g + 1) & 0xFFFFFFFF)
    u = ((x0 ^ x1) >> np.uint32(9)).astype(jnp.int32)
    u_ref[0] = u


def _gen_u():
    return pl.pallas_call(
        _gen_body,
        out_shape=jax.ShapeDtypeStruct((NROWS, SUB, LANE), jnp.int32),
        grid=(NROWS, ROW_CHUNKS),
        out_specs=pl.BlockSpec((1, BLK, LANE), lambda r, k: (r, k, 0)),
    )()


def _select_body(u_hbm, out_hbm, hist_v, buf_v, mg_v, cand_v, call_v,
                 tmp16_a, tmp16_b, sh_hist, sh_cand, sh_cnt, sh_binfo):
    c = lax.axis_index("c")
    s = lax.axis_index("s")
    lp = s // 4
    q = s % 4
    row = c * 4 + lp

    zeros16 = jnp.zeros((16,), jnp.int32)
    ones16 = jnp.ones((16,), jnp.int32)
    pad16 = jnp.full((16,), PAD, jnp.int32)
    iota16 = lax.iota(jnp.int32, 16)

    def z_body(i, _):
        hist_v[pl.ds(i * 16, 16)] = zeros16
        return 0
    lax.fori_loop(0, HBINS // 16, z_body, 0)

    def h_chunk(ch, _):
        pltpu.sync_copy(u_hbm.at[row, pl.ds(q * QUARTER + ch * CHUNK, CHUNK)], buf_v)

        def h_vec(i, _):
            v = buf_v[pl.ds(i * 16, 16)]
            plsc.addupdate_scatter(hist_v, [v >> BIN_SHIFT], ones16)
            return 0
        lax.fori_loop(0, NVEC, h_vec, 0)
        return 0
    lax.fori_loop(0, NCHUNK, h_chunk, 0)

    pltpu.sync_copy(hist_v, sh_hist.at[lp, q])
    plsc.subcore_barrier()

    @pl.when(q == 0)
    def _():
        for qq in range(4):
            pltpu.sync_copy(sh_hist.at[lp, qq], mg_v.at[qq])

        def sb(i, carry):
            run, binb, cntl = carry
            v = (mg_v[0, pl.ds(i * 16, 16)] + mg_v[1, pl.ds(i * 16, 16)]
                 + mg_v[2, pl.ds(i * 16, 16)] + mg_v[3, pl.ds(i * 16, 16)])
            tot = jnp.sum(v)
            cum = plsc.cumsum(v)
            m = (run + cum) >= LEN_KEEP
            lane = plsc.all_reduce_ffs(m)
            lane = lane if getattr(lane, "ndim", 0) == 0 else lane[0]
            excl = jnp.sum(jnp.where(iota16 < lane, v, 0))
            hit = jnp.logical_and(binb < 0, jnp.any(m))
            return (run + tot,
                    jnp.where(hit, i * 16 + lane, binb),
                    jnp.where(hit, run + excl, cntl))
        _, bsel, cnt_less = lax.fori_loop(0, HBINS // 16, sb, (0, -1, 0))

        kprime = LEN_KEEP - cnt_less
        info = jnp.where(iota16 == 0, bsel, jnp.where(iota16 == 1, kprime, 0))
        tmp16_a[...] = info
        pltpu.sync_copy(tmp16_a, sh_binfo.at[lp])

    plsc.subcore_barrier()

    pltpu.sync_copy(sh_binfo.at[lp], tmp16_b)
    binfo_v = tmp16_b[...]
    bsel = binfo_v[0]
    kprime = binfo_v[1]

    for i in range(QCAP // 16):
        cand_v[pl.ds(i * 16, 16)] = pad16

    def c_chunk(ch, cnt):
        pltpu.sync_copy(u_hbm.at[row, pl.ds(q * QUARTER + ch * CHUNK, CHUNK)], buf_v)

        def c_vec(i, cnt):
            v = buf_v[pl.ds(i * 16, 16)]
            m = (v >> BIN_SHIFT) == bsel
            nhit = plsc.all_reduce_population_count(m)
            nhit = nhit if getattr(nhit, "ndim", 0) == 0 else nhit[0]

            @pl.when(nhit > 0)
            def _():
                pos = q * QUARTER + ch * CHUNK + i * 16 + iota16
                keyv = ((v & LOW_MASK) << POS_BITS) | pos
                plsc.store_compressed(cand_v.at[pl.ds(cnt, 16)], keyv, mask=m)
            return cnt + nhit
        return lax.fori_loop(0, NVEC, c_vec, cnt)
    mycnt = lax.fori_loop(0, NCHUNK, c_chunk, 0)

    pltpu.sync_copy(cand_v, sh_cand.at[lp, q])
    tmp16_a[...] = jnp.where(iota16 == 0, mycnt, 0)
    pltpu.sync_copy(tmp16_a, sh_cnt.at[lp, q])
    plsc.subcore_barrier()

    @pl.when(q == 0)
    def _():
        def load_q(qq, _):
            pltpu.sync_copy(sh_cand.at[lp, qq], cand_v)
            pltpu.sync_copy(sh_cnt.at[lp, qq], tmp16_a)
            qcnt = tmp16_a[...][0]
            for j in range(QCAP // 16):
                vv = cand_v[pl.ds(j * 16, 16)]
                lanes = j * 16 + iota16
                call_v[pl.ds(qq * QCAP + j * 16, 16)] = jnp.where(lanes < qcnt, vv, PAD)
            return 0
        lax.fori_loop(0, 4, load_q, 0)

        def count_le(val):
            def cc(i, acc):
                v = call_v[pl.ds(i * 16, 16)]
                p = plsc.all_reduce_population_count(v <= val)
                p = p if getattr(p, "ndim", 0) == 0 else p[0]
                return acc + p
            return lax.fori_loop(0, 4 * QCAP // 16, cc, 0)

        def bisect(_, carry):
            lo, hi = carry
            mid = (lo + hi) >> 1
            ge = count_le(mid) >= kprime
            return (jnp.where(ge, lo, mid + 1), jnp.where(ge, mid, hi))
        lo, _hi = lax.fori_loop(0, BIN_SHIFT + POS_BITS, bisect,
                                (0, (1 << (BIN_SHIFT + POS_BITS)) - 1))

        outv = jnp.where(iota16 == 0, bsel, jnp.where(iota16 == 1, lo, 0))
        tmp16_a[...] = outv
        pltpu.sync_copy(tmp16_a, out_hbm.at[row])


def _select_thresholds(u_flat):
    mesh = plsc.VectorSubcoreMesh(core_axis_name="c", subcore_axis_name="s")
    f = functools.partial(
        pl.kernel,
        mesh=mesh,
        out_type=jax.ShapeDtypeStruct((NROWS, 16), jnp.int32),
        compiler_params=pltpu.CompilerParams(needs_layout_passes=False,
                                             use_tc_tiling_on_sc=False),
        scratch_types=[
            pltpu.VMEM((HBINS,), jnp.int32),
            pltpu.VMEM((CHUNK,), jnp.int32),
            pltpu.VMEM((4, HBINS), jnp.int32),
            pltpu.VMEM((QCAP,), jnp.int32),
            pltpu.VMEM((4 * QCAP,), jnp.int32),
            pltpu.VMEM((16,), jnp.int32),
            pltpu.VMEM((16,), jnp.int32),
            pltpu.VMEM_SHARED((4, 4, HBINS), jnp.int32),
            pltpu.VMEM_SHARED((4, 4, QCAP), jnp.int32),
            pltpu.VMEM_SHARED((4, 4, 16), jnp.int32),
            pltpu.VMEM_SHARED((4, 16), jnp.int32),
        ],
    )(_select_body)
    return f(u_flat)


def _mask_body(x_ref, u_ref, chm_ref, thr_ref, o_ref):
    r = pl.program_id(0)
    k = pl.program_id(1)
    bsel = thr_ref[r, 0]
    kbound = thr_ref[r, 1]
    u = u_ref[0]
    top = u >> BIN_SHIFT
    i0 = lax.broadcasted_iota(jnp.int32, (BLK, LANE), 0)
    i1 = lax.broadcasted_iota(jnp.int32, (BLK, LANE), 1)
    pos = k * BLK_ELEMS + i0 * LANE + i1
    keyv = ((u & LOW_MASK) << POS_BITS) | pos
    keep = (top < bsel) | ((top == bsel) & (keyv <= kbound))
    o_ref[0] = jnp.where(keep, x_ref[0] * chm_ref[...], 0.0)


def _apply_mask(xr, u, chm, thr):
    return pl.pallas_call(
        _mask_body,
        out_shape=jax.ShapeDtypeStruct((NROWS, SUB, LANE), jnp.float32),
        grid=(NROWS, ROW_CHUNKS),
        in_specs=[
            pl.BlockSpec((1, BLK, LANE), lambda r, k: (r, k, 0)),
            pl.BlockSpec((1, BLK, LANE), lambda r, k: (r, k, 0)),
            pl.BlockSpec((BLK, LANE), lambda r, k: (k, 0)),
            pl.BlockSpec(memory_space=pltpu.SMEM),
        ],
        out_specs=pl.BlockSpec((1, BLK, LANE), lambda r, k: (r, k, 0)),
    )(xr, u, chm, thr)


def kernel(x):
    xr = x.reshape(NROWS, SUB, LANE)
    u = _gen_u()
    thr = _select_thresholds(u.reshape(NROWS, L))
    out = _apply_mask(xr, u, jnp.asarray(_CHM), thr)
    return out.reshape(x.shape)


# speedup vs baseline: 22.7817x; 1.4317x over previous
import functools

import numpy as np
import jax
import jax.numpy as jnp
from jax import lax
from jax.experimental import pallas as pl
from jax.experimental.pallas import tpu as pltpu
from jax.experimental.pallas import tpu_sc as plsc

NROWS, CH, HH, WW = 8, 768, 24, 24
L = CH * HH * WW
MASK_RATIO = 0.3
LEN_KEEP = int(L * (1 - MASK_RATIO))
LANE = 128
SUB = L // LANE
ROW_CHUNKS = 8
BLK = SUB // ROW_CHUNKS
BLK_ELEMS = BLK * LANE

QUARTER = L // 4
NCHUNK = 8
CHUNK = QUARTER // NCHUNK
NVEC = CHUNK // 16
SUBCH = CHUNK // LANE
BIN_SHIFT = 10
HBINS = 1 << (23 - BIN_SHIFT)
LOW_MASK = (1 << BIN_SHIFT) - 1
POS_BITS = 19
QCAP = 64
PAD = 0x7FFFFFFF

_ROT_A = (13, 15, 26, 6)
_ROT_B = (17, 29, 16, 24)


def _np_threefry2x32(k0, k1, x0, x1):
    def rotl(x, r):
        return ((x << np.uint32(r)) | (x >> np.uint32(32 - r))).astype(np.uint32)

    k0, k1 = np.uint32(k0), np.uint32(k1)
    k2 = np.uint32(k0 ^ k1 ^ np.uint32(0x1BD11BDA))
    ks = (k0, k1, k2)
    x0 = (x0 + k0).astype(np.uint32)
    x1 = (x1 + k1).astype(np.uint32)
    for g in range(5):
        for r in (_ROT_A if g % 2 == 0 else _ROT_B):
            x0 = (x0 + x1).astype(np.uint32)
            x1 = rotl(x1, r)
            x1 = (x1 ^ x0).astype(np.uint32)
        x0 = (x0 + ks[(g + 1) % 3]).astype(np.uint32)
        x1 = (x1 + ks[(g + 2) % 3] + np.uint32(g + 1)).astype(np.uint32)
    return x0, x1


_FK0, _FK1 = _np_threefry2x32(
    np.uint32(0), np.uint32(42), np.array([0], np.uint32), np.array([1], np.uint32)
)
K0 = int(_FK0[0])
K1 = int(_FK1[0])
K2 = int(np.uint32(K0) ^ np.uint32(K1) ^ np.uint32(0x1BD11BDA))
_KS = (K0, K1, K2)

_ch_idx = np.asarray(np.random.default_rng(0).choice(CH, size=int(CH * 0.2), replace=False))
_chm = np.ones((CH,), np.float32)
_chm[_ch_idx] = 0.0
_CHM = np.repeat(_chm, HH * WW).reshape(SUB, LANE)


def _gen_body(u_ref):
    r = pl.program_id(0)
    k = pl.program_id(1)
    base = r * L + k * BLK_ELEMS
    i0 = lax.broadcasted_iota(jnp.int32, (BLK, LANE), 0)
    i1 = lax.broadcasted_iota(jnp.int32, (BLK, LANE), 1)
    cnt = (base + i0 * LANE + i1).astype(jnp.uint32)
    x0 = jnp.full((BLK, LANE), np.uint32(K0), jnp.uint32)
    x1 = cnt + np.uint32(K1)
    for g in range(5):
        for rot in (_ROT_A if g % 2 == 0 else _ROT_B):
            x0 = x0 + x1
            x1 = (x1 << np.uint32(rot)) | (x1 >> np.uint32(32 - rot))
            x1 = x1 ^ x0
        x0 = x0 + np.uint32(_KS[(g + 1) % 3])
        x1 = x1 + np.uint32((_KS[(g + 2) % 3] + g + 1) & 0xFFFFFFFF)
    u = ((x0 ^ x1) >> np.uint32(9)).astype(jnp.int32)
    u_ref[0] = u


def _gen_u():
    return pl.pallas_call(
        _gen_body,
        out_shape=jax.ShapeDtypeStruct((NROWS, SUB, LANE), jnp.int32),
        grid=(NROWS, ROW_CHUNKS),
        out_specs=pl.BlockSpec((1, BLK, LANE), lambda r, k: (r, k, 0)),
    )()


def _select_body(u_hbm, out_hbm, hist_v, buf_v, mg_v, cand_v, call_v,
                 tmp16_a, tmp16_b, sh_hist, sh_cand, sh_cnt, sh_binfo):
    c = lax.axis_index("c")
    s = lax.axis_index("s")
    lp = s // 4
    q = s % 4
    row = c * 4 + lp

    zeros16 = jnp.zeros((16,), jnp.int32)
    ones16 = jnp.ones((16,), jnp.int32)
    pad16 = jnp.full((16,), PAD, jnp.int32)
    iota16 = lax.iota(jnp.int32, 16)

    @plsc.parallel_loop(0, HBINS // 16, unroll=8)
    def _(i):
        hist_v[pl.ds(i * 16, 16)] = zeros16

    def h_chunk(ch, _):
        pltpu.sync_copy(
            u_hbm.at[row, pl.ds((q * NCHUNK + ch) * SUBCH, SUBCH), :], buf_v)

        @plsc.parallel_loop(0, NVEC, unroll=8)
        def _(i):
            v = buf_v[i >> 3, pl.ds((i & 7) * 16, 16)]
            plsc.addupdate_scatter(hist_v, [v >> BIN_SHIFT], ones16)
        return 0
    lax.fori_loop(0, NCHUNK, h_chunk, 0)

    pltpu.sync_copy(hist_v, sh_hist.at[lp, q])
    plsc.subcore_barrier()

    @pl.when(q == 0)
    def _():
        for qq in range(4):
            pltpu.sync_copy(sh_hist.at[lp, qq], mg_v.at[qq])

        def sb(i, carry):
            run, binb, cntl = carry
            v = (mg_v[0, pl.ds(i * 16, 16)] + mg_v[1, pl.ds(i * 16, 16)]
                 + mg_v[2, pl.ds(i * 16, 16)] + mg_v[3, pl.ds(i * 16, 16)])
            tot = jnp.sum(v)
            cum = plsc.cumsum(v)
            m = (run + cum) >= LEN_KEEP
            lane = plsc.all_reduce_ffs(m)
            lane = lane if getattr(lane, "ndim", 0) == 0 else lane[0]
            excl = jnp.sum(jnp.where(iota16 < lane, v, 0))
            hit = jnp.logical_and(binb < 0, jnp.any(m))
            return (run + tot,
                    jnp.where(hit, i * 16 + lane, binb),
                    jnp.where(hit, run + excl, cntl))
        _, bsel, cnt_less = lax.fori_loop(0, HBINS // 16, sb, (0, -1, 0))

        kprime = LEN_KEEP - cnt_less
        info = jnp.where(iota16 == 0, bsel, jnp.where(iota16 == 1, kprime, 0))
        tmp16_a[...] = info
        pltpu.sync_copy(tmp16_a, sh_binfo.at[lp])

    plsc.subcore_barrier()

    pltpu.sync_copy(sh_binfo.at[lp], tmp16_b)
    binfo_v = tmp16_b[...]
    bsel = binfo_v[0]
    kprime = binfo_v[1]

    for i in range(QCAP // 16):
        cand_v[pl.ds(i * 16, 16)] = pad16

    def c_chunk(ch, cnt):
        pltpu.sync_copy(
            u_hbm.at[row, pl.ds((q * NCHUNK + ch) * SUBCH, SUBCH), :], buf_v)

        def c_vec(i, cnt):
            v = buf_v[i >> 3, pl.ds((i & 7) * 16, 16)]
            m = (v >> BIN_SHIFT) == bsel
            pos = q * QUARTER + ch * CHUNK + i * 16 + iota16
            keyv = ((v & LOW_MASK) << POS_BITS) | pos
            plsc.store_compressed(cand_v.at[pl.ds(cnt, 16)], keyv, mask=m)
            nhit = plsc.all_reduce_population_count(m)
            nhit = nhit if getattr(nhit, "ndim", 0) == 0 else nhit[0]
            return cnt + nhit
        return plsc.parallel_loop(0, NVEC, unroll=4, carry=cnt)(c_vec)
    mycnt = lax.fori_loop(0, NCHUNK, c_chunk, 0)

    pltpu.sync_copy(cand_v, sh_cand.at[lp, q])
    tmp16_a[...] = jnp.where(iota16 == 0, mycnt, 0)
    pltpu.sync_copy(tmp16_a, sh_cnt.at[lp, q])
    plsc.subcore_barrier()

    @pl.when(q == 0)
    def _():
        def load_q(qq, _):
            pltpu.sync_copy(sh_cand.at[lp, qq], cand_v)
            pltpu.sync_copy(sh_cnt.at[lp, qq], tmp16_a)
            qcnt = tmp16_a[...][0]
            for j in range(QCAP // 16):
                vv = cand_v[pl.ds(j * 16, 16)]
                lanes = j * 16 + iota16
                call_v[pl.ds(qq * QCAP + j * 16, 16)] = jnp.where(lanes < qcnt, vv, PAD)
            return 0
        lax.fori_loop(0, 4, load_q, 0)

        def count_le(val):
            def cc(i, acc):
                v = call_v[pl.ds(i * 16, 16)]
                p = plsc.all_reduce_population_count(v <= val)
                p = p if getattr(p, "ndim", 0) == 0 else p[0]
                return acc + p
            return lax.fori_loop(0, 4 * QCAP // 16, cc, 0)

        def bisect(_, carry):
            lo, hi = carry
            mid = (lo + hi) >> 1
            ge = count_le(mid) >= kprime
            return (jnp.where(ge, lo, mid + 1), jnp.where(ge, mid, hi))
        lo, _hi = lax.fori_loop(0, BIN_SHIFT + POS_BITS, bisect,
                                (0, (1 << (BIN_SHIFT + POS_BITS)) - 1))

        outv = jnp.where(iota16 == 0, bsel, jnp.where(iota16 == 1, lo, 0))
        tmp16_a[...] = outv
        pltpu.sync_copy(tmp16_a, out_hbm.at[row])


def _select_thresholds(u_flat):
    mesh = plsc.VectorSubcoreMesh(core_axis_name="c", subcore_axis_name="s")
    f = functools.partial(
        pl.kernel,
        mesh=mesh,
        out_type=jax.ShapeDtypeStruct((NROWS, 16), jnp.int32),
        compiler_params=pltpu.CompilerParams(needs_layout_passes=False,
                                             use_tc_tiling_on_sc=False),
        scratch_types=[
            pltpu.VMEM((HBINS,), jnp.int32),
            pltpu.VMEM((SUBCH, LANE), jnp.int32),
            pltpu.VMEM((4, HBINS), jnp.int32),
            pltpu.VMEM((QCAP,), jnp.int32),
            pltpu.VMEM((4 * QCAP,), jnp.int32),
            pltpu.VMEM((16,), jnp.int32),
            pltpu.VMEM((16,), jnp.int32),
            pltpu.VMEM_SHARED((4, 4, HBINS), jnp.int32),
            pltpu.VMEM_SHARED((4, 4, QCAP), jnp.int32),
            pltpu.VMEM_SHARED((4, 4, 16), jnp.int32),
            pltpu.VMEM_SHARED((4, 16), jnp.int32),
        ],
    )(_select_body)
    return f(u_flat)


def _mask_body(x_ref, u_ref, chm_ref, thr_ref, o_ref):
    r = pl.program_id(0)
    k = pl.program_id(1)
    bsel = thr_ref[r, 0]
    kbound = thr_ref[r, 1]
    u = u_ref[0]
    top = u >> BIN_SHIFT
    i0 = lax.broadcasted_iota(jnp.int32, (BLK, LANE), 0)
    i1 = lax.broadcasted_iota(jnp.int32, (BLK, LANE), 1)
    pos = k * BLK_ELEMS + i0 * LANE + i1
    keyv = ((u & LOW_MASK) << POS_BITS) | pos
    keep = (top < bsel) | ((top == bsel) & (keyv <= kbound))
    o_ref[0] = jnp.where(keep, x_ref[0] * chm_ref[...], 0.0)


def _apply_mask(xr, u, chm, thr):
    return pl.pallas_call(
        _mask_body,
        out_shape=jax.ShapeDtypeStruct((NROWS, SUB, LANE), jnp.float32),
        grid=(NROWS, ROW_CHUNKS),
        in_specs=[
            pl.BlockSpec((1, BLK, LANE), lambda r, k: (r, k, 0)),
            pl.BlockSpec((1, BLK, LANE), lambda r, k: (r, k, 0)),
            pl.BlockSpec((BLK, LANE), lambda r, k: (k, 0)),
            pl.BlockSpec(memory_space=pltpu.SMEM),
        ],
        out_specs=pl.BlockSpec((1, BLK, LANE), lambda r, k: (r, k, 0)),
    )(xr, u, chm, thr)


def kernel(x):
    xr = x.reshape(NROWS, SUB, LANE)
    u = _gen_u()
    thr = _select_thresholds(u)
    out = _apply_mask(xr, u, jnp.asarray(_CHM), thr)
    return out.reshape(x.shape)


# speedup vs baseline: 39.5686x; 1.7369x over previous
import functools

import numpy as np
import jax
import jax.numpy as jnp
from jax import lax
from jax.experimental import pallas as pl
from jax.experimental.pallas import tpu as pltpu
from jax.experimental.pallas import tpu_sc as plsc

NROWS, CD, HH, WW = 8, 768, 24, 24
HW = HH * WW
L = CD * HW
MASK_RATIO = 0.3
LEN_KEEP = int(L * (1 - MASK_RATIO))
ROW_CHUNKS = 8
HWB = HW // ROW_CHUNKS

QUARTER = L // 4
Q_HW = HW // 4
NCHUNK = 8
CH_HW = Q_HW // NCHUNK
CHUNK = CH_HW * CD
VPH = CD // 16
NVEC = CH_HW * VPH
BIN_SHIFT = 10
HBINS = 1 << (23 - BIN_SHIFT)
LOW_MASK = (1 << BIN_SHIFT) - 1
POS_BITS = 19
QCAP = 64
PAD = 0x7FFFFFFF

_ROT_A = (13, 15, 26, 6)
_ROT_B = (17, 29, 16, 24)


def _np_threefry2x32(k0, k1, x0, x1):
    def rotl(x, r):
        return ((x << np.uint32(r)) | (x >> np.uint32(32 - r))).astype(np.uint32)

    k0, k1 = np.uint32(k0), np.uint32(k1)
    k2 = np.uint32(k0 ^ k1 ^ np.uint32(0x1BD11BDA))
    ks = (k0, k1, k2)
    x0 = (x0 + k0).astype(np.uint32)
    x1 = (x1 + k1).astype(np.uint32)
    for g in range(5):
        for r in (_ROT_A if g % 2 == 0 else _ROT_B):
            x0 = (x0 + x1).astype(np.uint32)
            x1 = rotl(x1, r)
            x1 = (x1 ^ x0).astype(np.uint32)
        x0 = (x0 + ks[(g + 1) % 3]).astype(np.uint32)
        x1 = (x1 + ks[(g + 2) % 3] + np.uint32(g + 1)).astype(np.uint32)
    return x0, x1


_FK0, _FK1 = _np_threefry2x32(
    np.uint32(0), np.uint32(42), np.array([0], np.uint32), np.array([1], np.uint32)
)
K0 = int(_FK0[0])
K1 = int(_FK1[0])
K2 = int(np.uint32(K0) ^ np.uint32(K1) ^ np.uint32(0x1BD11BDA))
_KS = (K0, K1, K2)

_ch_idx = np.asarray(np.random.default_rng(0).choice(CD, size=int(CD * 0.2), replace=False))
_CHM = np.ones((1, CD), np.float32)
_CHM[0, _ch_idx] = 0.0


def _gen_body(u_ref):
    r = pl.program_id(0)
    k = pl.program_id(1)
    i0 = lax.broadcasted_iota(jnp.int32, (HWB, CD), 0)
    i1 = lax.broadcasted_iota(jnp.int32, (HWB, CD), 1)
    cnt = (r * L + i1 * HW + k * HWB + i0).astype(jnp.uint32)
    x0 = jnp.full((HWB, CD), np.uint32(K0), jnp.uint32)
    x1 = cnt + np.uint32(K1)
    for g in range(5):
        for rot in (_ROT_A if g % 2 == 0 else _ROT_B):
            x0 = x0 + x1
            x1 = (x1 << np.uint32(rot)) | (x1 >> np.uint32(32 - rot))
            x1 = x1 ^ x0
        x0 = x0 + np.uint32(_KS[(g + 1) % 3])
        x1 = x1 + np.uint32((_KS[(g + 2) % 3] + g + 1) & 0xFFFFFFFF)
    u = ((x0 ^ x1) >> np.uint32(9)).astype(jnp.int32)
    u_ref[0] = u


def _gen_u():
    return pl.pallas_call(
        _gen_body,
        out_shape=jax.ShapeDtypeStruct((NROWS, HW, CD), jnp.int32),
        grid=(NROWS, ROW_CHUNKS),
        out_specs=pl.BlockSpec((1, HWB, CD), lambda r, k: (r, k, 0)),
    )()


def _select_body(u_hbm, out_hbm, hist_v, buf_v, mg_v, cand_v, call_v,
                 tmp16_a, tmp16_b, sh_hist, sh_cand, sh_cnt, sh_binfo):
    c = lax.axis_index("c")
    s = lax.axis_index("s")
    lp = s // 4
    q = s % 4
    row = c * 4 + lp

    zeros16 = jnp.zeros((16,), jnp.int32)
    ones16 = jnp.ones((16,), jnp.int32)
    pad16 = jnp.full((16,), PAD, jnp.int32)
    iota16 = lax.iota(jnp.int32, 16)

    @plsc.parallel_loop(0, HBINS // 16, unroll=8)
    def _(i):
        hist_v[pl.ds(i * 16, 16)] = zeros16

    def h_chunk(ch, _):
        pltpu.sync_copy(
            u_hbm.at[row, pl.ds((q * NCHUNK + ch) * CH_HW, CH_HW), :], buf_v)

        @plsc.parallel_loop(0, NVEC, unroll=8)
        def _(i):
            v = buf_v[i // VPH, pl.ds((i % VPH) * 16, 16)]
            plsc.addupdate_scatter(hist_v, [v >> BIN_SHIFT], ones16)
        return 0
    lax.fori_loop(0, NCHUNK, h_chunk, 0)

    pltpu.sync_copy(hist_v, sh_hist.at[lp, q])
    plsc.subcore_barrier()

    @pl.when(q == 0)
    def _():
        for qq in range(4):
            pltpu.sync_copy(sh_hist.at[lp, qq], mg_v.at[qq])

        def sb(i, carry):
            run, binb, cntl = carry
            v = (mg_v[0, pl.ds(i * 16, 16)] + mg_v[1, pl.ds(i * 16, 16)]
                 + mg_v[2, pl.ds(i * 16, 16)] + mg_v[3, pl.ds(i * 16, 16)])
            tot = jnp.sum(v)
            cum = plsc.cumsum(v)
            m = (run + cum) >= LEN_KEEP
            lane = plsc.all_reduce_ffs(m)
            lane = lane if getattr(lane, "ndim", 0) == 0 else lane[0]
            excl = jnp.sum(jnp.where(iota16 < lane, v, 0))
            hit = jnp.logical_and(binb < 0, jnp.any(m))
            return (run + tot,
                    jnp.where(hit, i * 16 + lane, binb),
                    jnp.where(hit, run + excl, cntl))
        _, bsel, cnt_less = lax.fori_loop(0, HBINS // 16, sb, (0, -1, 0))

        kprime = LEN_KEEP - cnt_less
        info = jnp.where(iota16 == 0, bsel, jnp.where(iota16 == 1, kprime, 0))
        tmp16_a[...] = info
        pltpu.sync_copy(tmp16_a, sh_binfo.at[lp])

    plsc.subcore_barrier()

    pltpu.sync_copy(sh_binfo.at[lp], tmp16_b)
    binfo_v = tmp16_b[...]
    bsel = binfo_v[0]
    kprime = binfo_v[1]

    for i in range(QCAP // 16):
        cand_v[pl.ds(i * 16, 16)] = pad16

    def c_chunk(ch, cnt):
        pltpu.sync_copy(
            u_hbm.at[row, pl.ds((q * NCHUNK + ch) * CH_HW, CH_HW), :], buf_v)

        def c_vec(i, cnt):
            v = buf_v[i // VPH, pl.ds((i % VPH) * 16, 16)]
            m = (v >> BIN_SHIFT) == bsel
            hw = (q * NCHUNK + ch) * CH_HW + i // VPH
            pos = ((i % VPH) * 16 + iota16) * HW + hw
            keyv = ((v & LOW_MASK) << POS_BITS) | pos
            plsc.store_compressed(cand_v.at[pl.ds(cnt, 16)], keyv, mask=m)
            nhit = plsc.all_reduce_population_count(m)
            nhit = nhit if getattr(nhit, "ndim", 0) == 0 else nhit[0]
            return cnt + nhit
        return plsc.parallel_loop(0, NVEC, unroll=4, carry=cnt)(c_vec)
    mycnt = lax.fori_loop(0, NCHUNK, c_chunk, 0)

    pltpu.sync_copy(cand_v, sh_cand.at[lp, q])
    tmp16_a[...] = jnp.where(iota16 == 0, mycnt, 0)
    pltpu.sync_copy(tmp16_a, sh_cnt.at[lp, q])
    plsc.subcore_barrier()

    @pl.when(q == 0)
    def _():
        def load_q(qq, _):
            pltpu.sync_copy(sh_cand.at[lp, qq], cand_v)
            pltpu.sync_copy(sh_cnt.at[lp, qq], tmp16_a)
            qcnt = tmp16_a[...][0]
            for j in range(QCAP // 16):
                vv = cand_v[pl.ds(j * 16, 16)]
                lanes = j * 16 + iota16
                call_v[pl.ds(qq * QCAP + j * 16, 16)] = jnp.where(lanes < qcnt, vv, PAD)
            return 0
        lax.fori_loop(0, 4, load_q, 0)

        def count_le(val):
            def cc(i, acc):
                v = call_v[pl.ds(i * 16, 16)]
                p = plsc.all_reduce_population_count(v <= val)
                p = p if getattr(p, "ndim", 0) == 0 else p[0]
                return acc + p
            return lax.fori_loop(0, 4 * QCAP // 16, cc, 0)

        def bisect(_, carry):
            lo, hi = carry
            mid = (lo + hi) >> 1
            ge = count_le(mid) >= kprime
            return (jnp.where(ge, lo, mid + 1), jnp.where(ge, mid, hi))
        lo, _hi = lax.fori_loop(0, BIN_SHIFT + POS_BITS, bisect,
                                (0, (1 << (BIN_SHIFT + POS_BITS)) - 1))

        outv = jnp.where(iota16 == 0, bsel, jnp.where(iota16 == 1, lo, 0))
        tmp16_a[...] = outv
        pltpu.sync_copy(tmp16_a, out_hbm.at[row])


def _select_thresholds(u):
    mesh = plsc.VectorSubcoreMesh(core_axis_name="c", subcore_axis_name="s")
    f = functools.partial(
        pl.kernel,
        mesh=mesh,
        out_type=jax.ShapeDtypeStruct((NROWS, 16), jnp.int32),
        compiler_params=pltpu.CompilerParams(needs_layout_passes=False,
                                             use_tc_tiling_on_sc=False),
        scratch_types=[
            pltpu.VMEM((HBINS,), jnp.int32),
            pltpu.VMEM((CH_HW, CD), jnp.int32),
            pltpu.VMEM((4, HBINS), jnp.int32),
            pltpu.VMEM((QCAP,), jnp.int32),
            pltpu.VMEM((4 * QCAP,), jnp.int32),
            pltpu.VMEM((16,), jnp.int32),
            pltpu.VMEM((16,), jnp.int32),
            pltpu.VMEM_SHARED((4, 4, HBINS), jnp.int32),
            pltpu.VMEM_SHARED((4, 4, QCAP), jnp.int32),
            pltpu.VMEM_SHARED((4, 4, 16), jnp.int32),
            pltpu.VMEM_SHARED((4, 16), jnp.int32),
        ],
    )(_select_body)
    return f(u)


def _mask_body(x_ref, u_ref, chm_ref, thr_ref, o_ref):
    r = pl.program_id(0)
    k = pl.program_id(1)
    bsel = thr_ref[r, 0]
    kbound = thr_ref[r, 1]
    u = u_ref[0]
    top = u >> BIN_SHIFT
    i0 = lax.broadcasted_iota(jnp.int32, (HWB, CD), 0)
    i1 = lax.broadcasted_iota(jnp.int32, (HWB, CD), 1)
    pos = i1 * HW + k * HWB + i0
    keyv = ((u & LOW_MASK) << POS_BITS) | pos
    keep = (top < bsel) | ((top == bsel) & (keyv <= kbound))
    o_ref[0] = jnp.where(keep, x_ref[0] * chm_ref[...], 0.0)


def _apply_mask(xt, u, chm, thr):
    return pl.pallas_call(
        _mask_body,
        out_shape=jax.ShapeDtypeStruct((NROWS, HW, CD), jnp.float32),
        grid=(NROWS, ROW_CHUNKS),
        in_specs=[
            pl.BlockSpec((1, HWB, CD), lambda r, k: (r, k, 0)),
            pl.BlockSpec((1, HWB, CD), lambda r, k: (r, k, 0)),
            pl.BlockSpec((1, CD), lambda r, k: (0, 0)),
            pl.BlockSpec(memory_space=pltpu.SMEM),
        ],
        out_specs=pl.BlockSpec((1, HWB, CD), lambda r, k: (r, k, 0)),
    )(xt, u, chm, thr)


def kernel(x):
    xt = x.transpose(0, 2, 3, 1).reshape(NROWS, HW, CD)
    u = _gen_u()
    thr = _select_thresholds(u)
    out = _apply_mask(xt, u, jnp.asarray(_CHM), thr)
    return out.reshape(NROWS, HH, WW, CD).transpose(0, 3, 1, 2)


# speedup vs baseline: 41.0778x; 1.0381x over previous
import functools

import numpy as np
import jax
import jax.numpy as jnp
from jax import lax
from jax.experimental import pallas as pl
from jax.experimental.pallas import tpu as pltpu
from jax.experimental.pallas import tpu_sc as plsc

NROWS, CD, HH, WW = 8, 768, 24, 24
HW = HH * WW
L = CD * HW
MASK_RATIO = 0.3
LEN_KEEP = int(L * (1 - MASK_RATIO))
ROW_CHUNKS = 8
HWB = HW // ROW_CHUNKS

HROWS = NROWS // 2
E_HW = HW // 8
NCHUNK = 4
CH_HW = E_HW // NCHUNK
CHUNK = CH_HW * CD
VPH = CD // 16
NVEC = CH_HW * VPH
BIN_SHIFT = 10
HBINS = 1 << (23 - BIN_SHIFT)
LOW_MASK = (1 << BIN_SHIFT) - 1
POS_BITS = 19
QCAP = 64
PAD = 0x7FFFFFFF

_ROT_A = (13, 15, 26, 6)
_ROT_B = (17, 29, 16, 24)


def _np_threefry2x32(k0, k1, x0, x1):
    def rotl(x, r):
        return ((x << np.uint32(r)) | (x >> np.uint32(32 - r))).astype(np.uint32)

    k0, k1 = np.uint32(k0), np.uint32(k1)
    k2 = np.uint32(k0 ^ k1 ^ np.uint32(0x1BD11BDA))
    ks = (k0, k1, k2)
    x0 = (x0 + k0).astype(np.uint32)
    x1 = (x1 + k1).astype(np.uint32)
    for g in range(5):
        for r in (_ROT_A if g % 2 == 0 else _ROT_B):
            x0 = (x0 + x1).astype(np.uint32)
            x1 = rotl(x1, r)
            x1 = (x1 ^ x0).astype(np.uint32)
        x0 = (x0 + ks[(g + 1) % 3]).astype(np.uint32)
        x1 = (x1 + ks[(g + 2) % 3] + np.uint32(g + 1)).astype(np.uint32)
    return x0, x1


_FK0, _FK1 = _np_threefry2x32(
    np.uint32(0), np.uint32(42), np.array([0], np.uint32), np.array([1], np.uint32)
)
K0 = int(_FK0[0])
K1 = int(_FK1[0])
K2 = int(np.uint32(K0) ^ np.uint32(K1) ^ np.uint32(0x1BD11BDA))
_KS = (K0, K1, K2)

_ch_idx = np.asarray(np.random.default_rng(0).choice(CD, size=int(CD * 0.2), replace=False))
_CHM = np.ones((1, CD), np.float32)
_CHM[0, _ch_idx] = 0.0


def _gen_body(u_ref, *, row0):
    r = pl.program_id(0)
    k = pl.program_id(1)
    i0 = lax.broadcasted_iota(jnp.int32, (HWB, CD), 0)
    i1 = lax.broadcasted_iota(jnp.int32, (HWB, CD), 1)
    cnt = ((row0 + r) * L + i1 * HW + k * HWB + i0).astype(jnp.uint32)
    x0 = jnp.full((HWB, CD), np.uint32(K0), jnp.uint32)
    x1 = cnt + np.uint32(K1)
    for g in range(5):
        for rot in (_ROT_A if g % 2 == 0 else _ROT_B):
            x0 = x0 + x1
            x1 = (x1 << np.uint32(rot)) | (x1 >> np.uint32(32 - rot))
            x1 = x1 ^ x0
        x0 = x0 + np.uint32(_KS[(g + 1) % 3])
        x1 = x1 + np.uint32((_KS[(g + 2) % 3] + g + 1) & 0xFFFFFFFF)
    u = ((x0 ^ x1) >> np.uint32(9)).astype(jnp.int32)
    u_ref[0] = u


def _gen_u(row0):
    return pl.pallas_call(
        functools.partial(_gen_body, row0=row0),
        out_shape=jax.ShapeDtypeStruct((HROWS, HW, CD), jnp.int32),
        grid=(HROWS, ROW_CHUNKS),
        out_specs=pl.BlockSpec((1, HWB, CD), lambda r, k: (r, k, 0)),
    )()


def _select_body(u_hbm, out_hbm, hist_v, buf_v, mg_v, cand_v, call_v,
                 tmp16_a, tmp16_b, sh_hist, sh_cand, sh_cnt, sh_binfo):
    c = lax.axis_index("c")
    s = lax.axis_index("s")
    lp = s // 8
    q = s % 8
    row = c * 2 + lp

    zeros16 = jnp.zeros((16,), jnp.int32)
    ones16 = jnp.ones((16,), jnp.int32)
    pad16 = jnp.full((16,), PAD, jnp.int32)
    iota16 = lax.iota(jnp.int32, 16)

    @plsc.parallel_loop(0, HBINS // 16, unroll=8)
    def _(i):
        hist_v[pl.ds(i * 16, 16)] = zeros16

    def h_chunk(ch, _):
        pltpu.sync_copy(
            u_hbm.at[row, pl.ds((q * NCHUNK + ch) * CH_HW, CH_HW), :], buf_v)

        @plsc.parallel_loop(0, NVEC, unroll=8)
        def _(i):
            v = buf_v[i // VPH, pl.ds((i % VPH) * 16, 16)]
            plsc.addupdate_scatter(hist_v, [v >> BIN_SHIFT], ones16)
        return 0
    lax.fori_loop(0, NCHUNK, h_chunk, 0)

    pltpu.sync_copy(hist_v, sh_hist.at[lp, q])
    plsc.subcore_barrier()

    @pl.when(q == 0)
    def _():
        for qq in range(8):
            pltpu.sync_copy(sh_hist.at[lp, qq], mg_v.at[qq])

        def sb(i, carry):
            run, binb, cntl = carry
            v = (mg_v[0, pl.ds(i * 16, 16)] + mg_v[1, pl.ds(i * 16, 16)]
                 + mg_v[2, pl.ds(i * 16, 16)] + mg_v[3, pl.ds(i * 16, 16)]
                 + mg_v[4, pl.ds(i * 16, 16)] + mg_v[5, pl.ds(i * 16, 16)]
                 + mg_v[6, pl.ds(i * 16, 16)] + mg_v[7, pl.ds(i * 16, 16)])
            tot = jnp.sum(v)
            cum = plsc.cumsum(v)
            m = (run + cum) >= LEN_KEEP
            lane = plsc.all_reduce_ffs(m)
            lane = lane if getattr(lane, "ndim", 0) == 0 else lane[0]
            excl = jnp.sum(jnp.where(iota16 < lane, v, 0))
            hit = jnp.logical_and(binb < 0, jnp.any(m))
            return (run + tot,
                    jnp.where(hit, i * 16 + lane, binb),
                    jnp.where(hit, run + excl, cntl))
        _, bsel, cnt_less = lax.fori_loop(0, HBINS // 16, sb, (0, -1, 0))

        kprime = LEN_KEEP - cnt_less
        info = jnp.where(iota16 == 0, bsel, jnp.where(iota16 == 1, kprime, 0))
        tmp16_a[...] = info
        pltpu.sync_copy(tmp16_a, sh_binfo.at[lp])

    plsc.subcore_barrier()

    pltpu.sync_copy(sh_binfo.at[lp], tmp16_b)
    binfo_v = tmp16_b[...]
    bsel = binfo_v[0]
    kprime = binfo_v[1]

    for i in range(QCAP // 16):
        cand_v[pl.ds(i * 16, 16)] = pad16

    def c_chunk(ch, cnt):
        pltpu.sync_copy(
            u_hbm.at[row, pl.ds((q * NCHUNK + ch) * CH_HW, CH_HW), :], buf_v)

        def c_vec(i, cnt):
            v = buf_v[i // VPH, pl.ds((i % VPH) * 16, 16)]
            m = (v >> BIN_SHIFT) == bsel
            hw = (q * NCHUNK + ch) * CH_HW + i // VPH
            pos = ((i % VPH) * 16 + iota16) * HW + hw
            keyv = ((v & LOW_MASK) << POS_BITS) | pos
            plsc.store_compressed(cand_v.at[pl.ds(cnt, 16)], keyv, mask=m)
            nhit = plsc.all_reduce_population_count(m)
            nhit = nhit if getattr(nhit, "ndim", 0) == 0 else nhit[0]
            return cnt + nhit
        return plsc.parallel_loop(0, NVEC, unroll=4, carry=cnt)(c_vec)
    mycnt = lax.fori_loop(0, NCHUNK, c_chunk, 0)

    pltpu.sync_copy(cand_v, sh_cand.at[lp, q])
    tmp16_a[...] = jnp.where(iota16 == 0, mycnt, 0)
    pltpu.sync_copy(tmp16_a, sh_cnt.at[lp, q])
    plsc.subcore_barrier()

    @pl.when(q == 0)
    def _():
        def load_q(qq, _):
            pltpu.sync_copy(sh_cand.at[lp, qq], cand_v)
            pltpu.sync_copy(sh_cnt.at[lp, qq], tmp16_a)
            qcnt = tmp16_a[...][0]
            for j in range(QCAP // 16):
                vv = cand_v[pl.ds(j * 16, 16)]
                lanes = j * 16 + iota16
                call_v[pl.ds(qq * QCAP + j * 16, 16)] = jnp.where(lanes < qcnt, vv, PAD)
            return 0
        lax.fori_loop(0, 8, load_q, 0)

        def count_le(val):
            def cc(i, acc):
                v = call_v[pl.ds(i * 16, 16)]
                p = plsc.all_reduce_population_count(v <= val)
                p = p if getattr(p, "ndim", 0) == 0 else p[0]
                return acc + p
            return lax.fori_loop(0, 8 * QCAP // 16, cc, 0)

        def bisect(_, carry):
            lo, hi = carry
            mid = (lo + hi) >> 1
            ge = count_le(mid) >= kprime
            return (jnp.where(ge, lo, mid + 1), jnp.where(ge, mid, hi))
        lo, _hi = lax.fori_loop(0, BIN_SHIFT + POS_BITS, bisect,
                                (0, (1 << (BIN_SHIFT + POS_BITS)) - 1))

        outv = jnp.where(iota16 == 0, bsel, jnp.where(iota16 == 1, lo, 0))
        tmp16_a[...] = outv
        pltpu.sync_copy(tmp16_a, out_hbm.at[row])


def _select_thresholds(u):
    mesh = plsc.VectorSubcoreMesh(core_axis_name="c", subcore_axis_name="s")
    f = functools.partial(
        pl.kernel,
        mesh=mesh,
        out_type=jax.ShapeDtypeStruct((HROWS, 16), jnp.int32),
        compiler_params=pltpu.CompilerParams(needs_layout_passes=False,
                                             use_tc_tiling_on_sc=False),
        scratch_types=[
            pltpu.VMEM((HBINS,), jnp.int32),
            pltpu.VMEM((CH_HW, CD), jnp.int32),
            pltpu.VMEM((8, HBINS), jnp.int32),
            pltpu.VMEM((QCAP,), jnp.int32),
            pltpu.VMEM((8 * QCAP,), jnp.int32),
            pltpu.VMEM((16,), jnp.int32),
            pltpu.VMEM((16,), jnp.int32),
            pltpu.VMEM_SHARED((2, 8, HBINS), jnp.int32),
            pltpu.VMEM_SHARED((2, 8, QCAP), jnp.int32),
            pltpu.VMEM_SHARED((2, 8, 16), jnp.int32),
            pltpu.VMEM_SHARED((2, 16), jnp.int32),
        ],
    )(_select_body)
    return f(u)


def _mask_body(x_ref, u_ref, chm_ref, thr_ref, o_ref):
    r = pl.program_id(0)
    k = pl.program_id(1)
    bsel = thr_ref[r, 0]
    kbound = thr_ref[r, 1]
    u = u_ref[0]
    top = u >> BIN_SHIFT
    i0 = lax.broadcasted_iota(jnp.int32, (HWB, CD), 0)
    i1 = lax.broadcasted_iota(jnp.int32, (HWB, CD), 1)
    pos = i1 * HW + k * HWB + i0
    keyv = ((u & LOW_MASK) << POS_BITS) | pos
    keep = (top < bsel) | ((top == bsel) & (keyv <= kbound))
    o_ref[0] = jnp.where(keep, x_ref[0] * chm_ref[...], 0.0)


def _apply_mask_a(xt, u_a, chm, thr_a):
    return pl.pallas_call(
        _mask_body,
        out_shape=jax.ShapeDtypeStruct((NROWS, HW, CD), jnp.float32),
        grid=(HROWS, ROW_CHUNKS),
        in_specs=[
            pl.BlockSpec((1, HWB, CD), lambda r, k: (r, k, 0)),
            pl.BlockSpec((1, HWB, CD), lambda r, k: (r, k, 0)),
            pl.BlockSpec((1, CD), lambda r, k: (0, 0)),
            pl.BlockSpec(memory_space=pltpu.SMEM),
        ],
        out_specs=pl.BlockSpec((1, HWB, CD), lambda r, k: (r, k, 0)),
    )(xt, u_a, chm, thr_a)


def _mask_body_b(buf_ref, x_ref, u_ref, chm_ref, thr_ref, o_ref):
    del buf_ref
    _mask_body(x_ref, u_ref, chm_ref, thr_ref, o_ref)


def _apply_mask_b(buf, xt, u_b, chm, thr_b):
    return pl.pallas_call(
        _mask_body_b,
        out_shape=jax.ShapeDtypeStruct((NROWS, HW, CD), jnp.float32),
        grid=(HROWS, ROW_CHUNKS),
        in_specs=[
            pl.BlockSpec(memory_space=pl.ANY),
            pl.BlockSpec((1, HWB, CD), lambda r, k: (r + HROWS, k, 0)),
            pl.BlockSpec((1, HWB, CD), lambda r, k: (r, k, 0)),
            pl.BlockSpec((1, CD), lambda r, k: (0, 0)),
            pl.BlockSpec(memory_space=pltpu.SMEM),
        ],
        out_specs=pl.BlockSpec((1, HWB, CD), lambda r, k: (r + HROWS, k, 0)),
        input_output_aliases={0: 0},
    )(buf, xt, u_b, chm, thr_b)


def kernel(x):
    xt = x.transpose(0, 2, 3, 1).reshape(NROWS, HW, CD)
    chm = jnp.asarray(_CHM)
    u_a = _gen_u(0)
    thr_a = _select_thresholds(u_a)
    u_b = _gen_u(HROWS)
    thr_b = _select_thresholds(u_b)
    out = _apply_mask_a(xt, u_a, chm, thr_a)
    out = _apply_mask_b(out, xt, u_b, chm, thr_b)
    return out.reshape(NROWS, HH, WW, CD).transpose(0, 3, 1, 2)


# speedup vs baseline: 44.4779x; 1.0828x over previous
import functools

import numpy as np
import jax
import jax.numpy as jnp
from jax import lax
from jax.experimental import pallas as pl
from jax.experimental.pallas import tpu as pltpu
from jax.experimental.pallas import tpu_sc as plsc

NROWS, CD, HH, WW = 8, 768, 24, 24
HW = HH * WW
L = CD * HW
MASK_RATIO = 0.3
LEN_KEEP = int(L * (1 - MASK_RATIO))
ROW_CHUNKS = 8
HWB = HW // ROW_CHUNKS

HROWS = NROWS // 2
E_HW = HW // 8
NCHUNK = 4
CH_HW = E_HW // NCHUNK
CHUNK = CH_HW * CD
VPH = CD // 16
NVEC = CH_HW * VPH
BIN_SHIFT = 10
HBINS = 1 << (23 - BIN_SHIFT)
LOW_MASK = (1 << BIN_SHIFT) - 1
POS_BITS = 19
QCAP = 64
PAD = 0x7FFFFFFF

_ROT_A = (13, 15, 26, 6)
_ROT_B = (17, 29, 16, 24)


def _np_threefry2x32(k0, k1, x0, x1):
    def rotl(x, r):
        return ((x << np.uint32(r)) | (x >> np.uint32(32 - r))).astype(np.uint32)

    k0, k1 = np.uint32(k0), np.uint32(k1)
    k2 = np.uint32(k0 ^ k1 ^ np.uint32(0x1BD11BDA))
    ks = (k0, k1, k2)
    x0 = (x0 + k0).astype(np.uint32)
    x1 = (x1 + k1).astype(np.uint32)
    for g in range(5):
        for r in (_ROT_A if g % 2 == 0 else _ROT_B):
            x0 = (x0 + x1).astype(np.uint32)
            x1 = rotl(x1, r)
            x1 = (x1 ^ x0).astype(np.uint32)
        x0 = (x0 + ks[(g + 1) % 3]).astype(np.uint32)
        x1 = (x1 + ks[(g + 2) % 3] + np.uint32(g + 1)).astype(np.uint32)
    return x0, x1


_FK0, _FK1 = _np_threefry2x32(
    np.uint32(0), np.uint32(42), np.array([0], np.uint32), np.array([1], np.uint32)
)
K0 = int(_FK0[0])
K1 = int(_FK1[0])
K2 = int(np.uint32(K0) ^ np.uint32(K1) ^ np.uint32(0x1BD11BDA))
_KS = (K0, K1, K2)

_ch_idx = np.asarray(np.random.default_rng(0).choice(CD, size=int(CD * 0.2), replace=False))
_CHM = np.ones((1, CD), np.float32)
_CHM[0, _ch_idx] = 0.0


def _gen_body(u_ref, *, row0):
    r = pl.program_id(0)
    k = pl.program_id(1)
    i0 = lax.broadcasted_iota(jnp.int32, (HWB, CD), 0)
    i1 = lax.broadcasted_iota(jnp.int32, (HWB, CD), 1)
    cnt = ((row0 + r) * L + i1 * HW + k * HWB + i0).astype(jnp.uint32)
    x0 = jnp.full((HWB, CD), np.uint32(K0), jnp.uint32)
    x1 = cnt + np.uint32(K1)
    for g in range(5):
        for rot in (_ROT_A if g % 2 == 0 else _ROT_B):
            x0 = x0 + x1
            x1 = (x1 << np.uint32(rot)) | (x1 >> np.uint32(32 - rot))
            x1 = x1 ^ x0
        x0 = x0 + np.uint32(_KS[(g + 1) % 3])
        x1 = x1 + np.uint32((_KS[(g + 2) % 3] + g + 1) & 0xFFFFFFFF)
    u = ((x0 ^ x1) >> np.uint32(9)).astype(jnp.int32)
    u_ref[0] = u


def _gen_u(row0):
    return pl.pallas_call(
        functools.partial(_gen_body, row0=row0),
        out_shape=jax.ShapeDtypeStruct((HROWS, HW, CD), jnp.int32),
        grid=(HROWS, ROW_CHUNKS),
        out_specs=pl.BlockSpec((1, HWB, CD), lambda r, k: (r, k, 0)),
    )()


def _select_body(u_hbm, out_hbm, hist_v, buf_v, mg_v, cand_v, call_v,
                 tmp16_a, tmp16_b, sem, sh_hist, sh_cand, sh_cnt, sh_binfo):
    c = lax.axis_index("c")
    s = lax.axis_index("s")
    lp = s // 8
    q = s % 8
    row = c * 2 + lp

    zeros16 = jnp.zeros((16,), jnp.int32)
    ones16 = jnp.ones((16,), jnp.int32)
    pad16 = jnp.full((16,), PAD, jnp.int32)
    iota16 = lax.iota(jnp.int32, 16)

    @plsc.parallel_loop(0, HBINS // 16, unroll=8)
    def _(i):
        hist_v[pl.ds(i * 16, 16)] = zeros16

    def chunk_src(ch):
        return u_hbm.at[row, pl.ds((q * NCHUNK + ch) * CH_HW, CH_HW), :]

    pltpu.async_copy(chunk_src(0), buf_v.at[0], sem.at[0])
    for ch in range(NCHUNK):
        if ch + 1 < NCHUNK:
            pltpu.async_copy(chunk_src(ch + 1), buf_v.at[(ch + 1) % 2],
                             sem.at[(ch + 1) % 2])
        pltpu.make_async_copy(chunk_src(ch), buf_v.at[ch % 2],
                              sem.at[ch % 2]).wait()

        @plsc.parallel_loop(0, NVEC, unroll=8)
        def _(i, _b=ch % 2):
            v = buf_v[_b, i // VPH, pl.ds((i % VPH) * 16, 16)]
            plsc.addupdate_scatter(hist_v, [v >> BIN_SHIFT], ones16)

    pltpu.sync_copy(hist_v, sh_hist.at[lp, q])
    plsc.subcore_barrier()

    @pl.when(q == 0)
    def _():
        for qq in range(8):
            pltpu.sync_copy(sh_hist.at[lp, qq], mg_v.at[qq])

        def sb(i, carry):
            run, binb, cntl = carry
            v = (mg_v[0, pl.ds(i * 16, 16)] + mg_v[1, pl.ds(i * 16, 16)]
                 + mg_v[2, pl.ds(i * 16, 16)] + mg_v[3, pl.ds(i * 16, 16)]
                 + mg_v[4, pl.ds(i * 16, 16)] + mg_v[5, pl.ds(i * 16, 16)]
                 + mg_v[6, pl.ds(i * 16, 16)] + mg_v[7, pl.ds(i * 16, 16)])
            tot = jnp.sum(v)
            cum = plsc.cumsum(v)
            m = (run + cum) >= LEN_KEEP
            lane = plsc.all_reduce_ffs(m)
            lane = lane if getattr(lane, "ndim", 0) == 0 else lane[0]
            excl = jnp.sum(jnp.where(iota16 < lane, v, 0))
            hit = jnp.logical_and(binb < 0, jnp.any(m))
            return (run + tot,
                    jnp.where(hit, i * 16 + lane, binb),
                    jnp.where(hit, run + excl, cntl))
        _, bsel, cnt_less = lax.fori_loop(0, HBINS // 16, sb, (0, -1, 0))

        kprime = LEN_KEEP - cnt_less
        info = jnp.where(iota16 == 0, bsel, jnp.where(iota16 == 1, kprime, 0))
        tmp16_a[...] = info
        pltpu.sync_copy(tmp16_a, sh_binfo.at[lp])

    plsc.subcore_barrier()

    pltpu.sync_copy(sh_binfo.at[lp], tmp16_b)
    binfo_v = tmp16_b[...]
    bsel = binfo_v[0]
    kprime = binfo_v[1]

    for i in range(QCAP // 16):
        cand_v[pl.ds(i * 16, 16)] = pad16

    mycnt = jnp.int32(0)
    pltpu.async_copy(chunk_src(0), buf_v.at[0], sem.at[0])
    for ch in range(NCHUNK):
        if ch + 1 < NCHUNK:
            pltpu.async_copy(chunk_src(ch + 1), buf_v.at[(ch + 1) % 2],
                             sem.at[(ch + 1) % 2])
        pltpu.make_async_copy(chunk_src(ch), buf_v.at[ch % 2],
                              sem.at[ch % 2]).wait()

        def c_vec(i, cnt, _b=ch % 2, _ch=ch):
            v = buf_v[_b, i // VPH, pl.ds((i % VPH) * 16, 16)]
            m = (v >> BIN_SHIFT) == bsel
            hw = (q * NCHUNK + _ch) * CH_HW + i // VPH
            pos = ((i % VPH) * 16 + iota16) * HW + hw
            keyv = ((v & LOW_MASK) << POS_BITS) | pos
            plsc.store_compressed(cand_v.at[pl.ds(cnt, 16)], keyv, mask=m)
            nhit = plsc.all_reduce_population_count(m)
            nhit = nhit if getattr(nhit, "ndim", 0) == 0 else nhit[0]
            return cnt + nhit
        mycnt = plsc.parallel_loop(0, NVEC, unroll=4, carry=mycnt)(c_vec)

    pltpu.sync_copy(cand_v, sh_cand.at[lp, q])
    tmp16_a[...] = jnp.where(iota16 == 0, mycnt, 0)
    pltpu.sync_copy(tmp16_a, sh_cnt.at[lp, q])
    plsc.subcore_barrier()

    @pl.when(q == 0)
    def _():
        def load_q(qq, _):
            pltpu.sync_copy(sh_cand.at[lp, qq], cand_v)
            pltpu.sync_copy(sh_cnt.at[lp, qq], tmp16_a)
            qcnt = tmp16_a[...][0]
            for j in range(QCAP // 16):
                vv = cand_v[pl.ds(j * 16, 16)]
                lanes = j * 16 + iota16
                call_v[pl.ds(qq * QCAP + j * 16, 16)] = jnp.where(lanes < qcnt, vv, PAD)
            return 0
        lax.fori_loop(0, 8, load_q, 0)

        def count_le(val):
            def cc(i, acc):
                v = call_v[pl.ds(i * 16, 16)]
                p = plsc.all_reduce_population_count(v <= val)
                p = p if getattr(p, "ndim", 0) == 0 else p[0]
                return acc + p
            return lax.fori_loop(0, 8 * QCAP // 16, cc, 0)

        def bisect(_, carry):
            lo, hi = carry
            mid = (lo + hi) >> 1
            ge = count_le(mid) >= kprime
            return (jnp.where(ge, lo, mid + 1), jnp.where(ge, mid, hi))
        lo, _hi = lax.fori_loop(0, BIN_SHIFT + POS_BITS, bisect,
                                (0, (1 << (BIN_SHIFT + POS_BITS)) - 1))

        outv = jnp.where(iota16 == 0, bsel, jnp.where(iota16 == 1, lo, 0))
        tmp16_a[...] = outv
        pltpu.sync_copy(tmp16_a, out_hbm.at[row])


def _select_thresholds(u):
    mesh = plsc.VectorSubcoreMesh(core_axis_name="c", subcore_axis_name="s")
    f = functools.partial(
        pl.kernel,
        mesh=mesh,
        out_type=jax.ShapeDtypeStruct((HROWS, 16), jnp.int32),
        compiler_params=pltpu.CompilerParams(needs_layout_passes=False,
                                             use_tc_tiling_on_sc=False),
        scratch_types=[
            pltpu.VMEM((HBINS,), jnp.int32),
            pltpu.VMEM((2, CH_HW, CD), jnp.int32),
            pltpu.VMEM((8, HBINS), jnp.int32),
            pltpu.VMEM((QCAP,), jnp.int32),
            pltpu.VMEM((8 * QCAP,), jnp.int32),
            pltpu.VMEM((16,), jnp.int32),
            pltpu.VMEM((16,), jnp.int32),
            pltpu.SemaphoreType.DMA((2,)),
            pltpu.VMEM_SHARED((2, 8, HBINS), jnp.int32),
            pltpu.VMEM_SHARED((2, 8, QCAP), jnp.int32),
            pltpu.VMEM_SHARED((2, 8, 16), jnp.int32),
            pltpu.VMEM_SHARED((2, 16), jnp.int32),
        ],
    )(_select_body)
    return f(u)


def _mask_body(x_ref, u_ref, chm_ref, thr_ref, o_ref):
    r = pl.program_id(0)
    k = pl.program_id(1)
    bsel = thr_ref[r, 0]
    kbound = thr_ref[r, 1]
    u = u_ref[0]
    top = u >> BIN_SHIFT
    i0 = lax.broadcasted_iota(jnp.int32, (HWB, CD), 0)
    i1 = lax.broadcasted_iota(jnp.int32, (HWB, CD), 1)
    pos = i1 * HW + k * HWB + i0
    keyv = ((u & LOW_MASK) << POS_BITS) | pos
    keep = (top < bsel) | ((top == bsel) & (keyv <= kbound))
    o_ref[0] = jnp.where(keep, x_ref[0] * chm_ref[...], 0.0)


def _apply_mask_a(xt, u_a, chm, thr_a):
    return pl.pallas_call(
        _mask_body,
        out_shape=jax.ShapeDtypeStruct((NROWS, HW, CD), jnp.float32),
        grid=(HROWS, ROW_CHUNKS),
        in_specs=[
            pl.BlockSpec((1, HWB, CD), lambda r, k: (r, k, 0)),
            pl.BlockSpec((1, HWB, CD), lambda r, k: (r, k, 0)),
            pl.BlockSpec((1, CD), lambda r, k: (0, 0)),
            pl.BlockSpec(memory_space=pltpu.SMEM),
        ],
        out_specs=pl.BlockSpec((1, HWB, CD), lambda r, k: (r, k, 0)),
    )(xt, u_a, chm, thr_a)


def _mask_body_b(buf_ref, x_ref, u_ref, chm_ref, thr_ref, o_ref):
    del buf_ref
    _mask_body(x_ref, u_ref, chm_ref, thr_ref, o_ref)


def _apply_mask_b(buf, xt, u_b, chm, thr_b):
    return pl.pallas_call(
        _mask_body_b,
        out_shape=jax.ShapeDtypeStruct((NROWS, HW, CD), jnp.float32),
        grid=(HROWS, ROW_CHUNKS),
        in_specs=[
            pl.BlockSpec(memory_space=pl.ANY),
            pl.BlockSpec((1, HWB, CD), lambda r, k: (r + HROWS, k, 0)),
            pl.BlockSpec((1, HWB, CD), lambda r, k: (r, k, 0)),
            pl.BlockSpec((1, CD), lambda r, k: (0, 0)),
            pl.BlockSpec(memory_space=pltpu.SMEM),
        ],
        out_specs=pl.BlockSpec((1, HWB, CD), lambda r, k: (r + HROWS, k, 0)),
        input_output_aliases={0: 0},
    )(buf, xt, u_b, chm, thr_b)


def kernel(x):
    xt = x.transpose(0, 2, 3, 1).reshape(NROWS, HW, CD)
    chm = jnp.asarray(_CHM)
    u_a = _gen_u(0)
    thr_a = _select_thresholds(u_a)
    u_b = _gen_u(HROWS)
    thr_b = _select_thresholds(u_b)
    out = _apply_mask_a(xt, u_a, chm, thr_a)
    out = _apply_mask_b(out, xt, u_b, chm, thr_b)
    return out.reshape(NROWS, HH, WW, CD).transpose(0, 3, 1, 2)


# speedup vs baseline: 44.5273x; 1.0011x over previous
import functools

import numpy as np
import jax
import jax.numpy as jnp
from jax import lax
from jax.experimental import pallas as pl
from jax.experimental.pallas import tpu as pltpu
from jax.experimental.pallas import tpu_sc as plsc

NROWS, CD, HH, WW = 8, 768, 24, 24
HW = HH * WW
L = CD * HW
MASK_RATIO = 0.3
LEN_KEEP = int(L * (1 - MASK_RATIO))
ROW_CHUNKS = 8
HWB = HW // ROW_CHUNKS

HROWS = NROWS // 2
E_HW = HW // 8
NCHUNK = 4
CH_HW = E_HW // NCHUNK
CHUNK = CH_HW * CD
VPH = CD // 16
NVEC = CH_HW * VPH
BIN_SHIFT = 10
HBINS = 1 << (23 - BIN_SHIFT)
LOW_MASK = (1 << BIN_SHIFT) - 1
POS_BITS = 19
QCAP = 64
PAD = 0x7FFFFFFF

_ROT_A = (13, 15, 26, 6)
_ROT_B = (17, 29, 16, 24)


def _np_threefry2x32(k0, k1, x0, x1):
    def rotl(x, r):
        return ((x << np.uint32(r)) | (x >> np.uint32(32 - r))).astype(np.uint32)

    k0, k1 = np.uint32(k0), np.uint32(k1)
    k2 = np.uint32(k0 ^ k1 ^ np.uint32(0x1BD11BDA))
    ks = (k0, k1, k2)
    x0 = (x0 + k0).astype(np.uint32)
    x1 = (x1 + k1).astype(np.uint32)
    for g in range(5):
        for r in (_ROT_A if g % 2 == 0 else _ROT_B):
            x0 = (x0 + x1).astype(np.uint32)
            x1 = rotl(x1, r)
            x1 = (x1 ^ x0).astype(np.uint32)
        x0 = (x0 + ks[(g + 1) % 3]).astype(np.uint32)
        x1 = (x1 + ks[(g + 2) % 3] + np.uint32(g + 1)).astype(np.uint32)
    return x0, x1


_FK0, _FK1 = _np_threefry2x32(
    np.uint32(0), np.uint32(42), np.array([0], np.uint32), np.array([1], np.uint32)
)
K0 = int(_FK0[0])
K1 = int(_FK1[0])
K2 = int(np.uint32(K0) ^ np.uint32(K1) ^ np.uint32(0x1BD11BDA))
_KS = (K0, K1, K2)

_ch_idx = np.asarray(np.random.default_rng(0).choice(CD, size=int(CD * 0.2), replace=False))
_CHM = np.ones((1, CD), np.float32)
_CHM[0, _ch_idx] = 0.0


def _gen_body(u_ref, *, row0):
    r = pl.program_id(0)
    k = pl.program_id(1)
    i0 = lax.broadcasted_iota(jnp.int32, (HWB, CD), 0)
    i1 = lax.broadcasted_iota(jnp.int32, (HWB, CD), 1)
    cnt = ((row0 + r) * L + i1 * HW + k * HWB + i0).astype(jnp.uint32)
    x0 = jnp.full((HWB, CD), np.uint32(K0), jnp.uint32)
    x1 = cnt + np.uint32(K1)
    for g in range(5):
        for rot in (_ROT_A if g % 2 == 0 else _ROT_B):
            x0 = x0 + x1
            x1 = (x1 << np.uint32(rot)) | (x1 >> np.uint32(32 - rot))
            x1 = x1 ^ x0
        x0 = x0 + np.uint32(_KS[(g + 1) % 3])
        x1 = x1 + np.uint32((_KS[(g + 2) % 3] + g + 1) & 0xFFFFFFFF)
    u = ((x0 ^ x1) >> np.uint32(9)).astype(jnp.int32)
    u_ref[0] = u


def _gen_u(row0):
    return pl.pallas_call(
        functools.partial(_gen_body, row0=row0),
        out_shape=jax.ShapeDtypeStruct((HROWS, HW, CD), jnp.int32),
        grid=(HROWS, ROW_CHUNKS),
        out_specs=pl.BlockSpec((1, HWB, CD), lambda r, k: (r, k, 0)),
    )()


def _select_body(u_hbm, out_hbm, hist_v, buf_v, mg_v, cand_v, call_v,
                 tmp16_a, tmp16_b, sem, sh_hist, sh_cand, sh_cnt, sh_binfo):
    c = lax.axis_index("c")
    s = lax.axis_index("s")
    lp = s // 8
    q = s % 8
    row = c * 2 + lp

    zeros16 = jnp.zeros((16,), jnp.int32)
    ones16 = jnp.ones((16,), jnp.int32)
    pad16 = jnp.full((16,), PAD, jnp.int32)
    iota16 = lax.iota(jnp.int32, 16)

    @plsc.parallel_loop(0, HBINS // 16, unroll=8)
    def _(i):
        hist_v[pl.ds(i * 16, 16)] = zeros16

    def chunk_src(ch):
        return u_hbm.at[row, pl.ds((q * NCHUNK + ch) * CH_HW, CH_HW), :]

    pltpu.async_copy(chunk_src(0), buf_v.at[0], sem.at[0])
    for ch in range(NCHUNK):
        if ch + 1 < NCHUNK:
            pltpu.async_copy(chunk_src(ch + 1), buf_v.at[(ch + 1) % 2],
                             sem.at[(ch + 1) % 2])
        pltpu.make_async_copy(chunk_src(ch), buf_v.at[ch % 2],
                              sem.at[ch % 2]).wait()

        @plsc.parallel_loop(0, NVEC, unroll=8)
        def _(i, _b=ch % 2):
            v = buf_v[_b, i // VPH, pl.ds((i % VPH) * 16, 16)]
            plsc.addupdate_scatter(hist_v, [v >> BIN_SHIFT], ones16)

    pltpu.sync_copy(hist_v, sh_hist.at[lp, q])
    plsc.subcore_barrier()

    @pl.when(q == 0)
    def _():
        for qq in range(8):
            pltpu.sync_copy(sh_hist.at[lp, qq], mg_v.at[qq])

        def sb(i, carry):
            run, binb, cntl = carry
            v = (mg_v[0, pl.ds(i * 16, 16)] + mg_v[1, pl.ds(i * 16, 16)]
                 + mg_v[2, pl.ds(i * 16, 16)] + mg_v[3, pl.ds(i * 16, 16)]
                 + mg_v[4, pl.ds(i * 16, 16)] + mg_v[5, pl.ds(i * 16, 16)]
                 + mg_v[6, pl.ds(i * 16, 16)] + mg_v[7, pl.ds(i * 16, 16)])
            tot = jnp.sum(v)
            cum = plsc.cumsum(v)
            m = (run + cum) >= LEN_KEEP
            lane = plsc.all_reduce_ffs(m)
            lane = lane if getattr(lane, "ndim", 0) == 0 else lane[0]
            excl = jnp.sum(jnp.where(iota16 < lane, v, 0))
            hit = jnp.logical_and(binb < 0, jnp.any(m))
            return (run + tot,
                    jnp.where(hit, i * 16 + lane, binb),
                    jnp.where(hit, run + excl, cntl))
        _, bsel, cnt_less = lax.fori_loop(0, HBINS // 16, sb, (0, -1, 0))

        kprime = LEN_KEEP - cnt_less
        info = jnp.where(iota16 == 0, bsel, jnp.where(iota16 == 1, kprime, 0))
        tmp16_a[...] = info
        pltpu.sync_copy(tmp16_a, sh_binfo.at[lp])

    plsc.subcore_barrier()

    pltpu.sync_copy(sh_binfo.at[lp], tmp16_b)
    binfo_v = tmp16_b[...]
    bsel = binfo_v[0]
    kprime = binfo_v[1]

    for i in range(QCAP // 16):
        cand_v[pl.ds(i * 16, 16)] = pad16

    mycnt = jnp.int32(0)
    pltpu.async_copy(chunk_src(0), buf_v.at[0], sem.at[0])
    for ch in range(NCHUNK):
        if ch + 1 < NCHUNK:
            pltpu.async_copy(chunk_src(ch + 1), buf_v.at[(ch + 1) % 2],
                             sem.at[(ch + 1) % 2])
        pltpu.make_async_copy(chunk_src(ch), buf_v.at[ch % 2],
                              sem.at[ch % 2]).wait()

        def c_vec(i, cnt, _b=ch % 2, _ch=ch):
            v = buf_v[_b, i // VPH, pl.ds((i % VPH) * 16, 16)]
            m = (v >> BIN_SHIFT) == bsel
            hw = (q * NCHUNK + _ch) * CH_HW + i // VPH
            pos = ((i % VPH) * 16 + iota16) * HW + hw
            keyv = ((v & LOW_MASK) << POS_BITS) | pos
            plsc.store_compressed(cand_v.at[pl.ds(cnt, 16)], keyv, mask=m)
            nhit = plsc.all_reduce_population_count(m)
            nhit = nhit if getattr(nhit, "ndim", 0) == 0 else nhit[0]
            return cnt + nhit
        mycnt = plsc.parallel_loop(0, NVEC, unroll=8, carry=mycnt)(c_vec)

    pltpu.sync_copy(cand_v, sh_cand.at[lp, q])
    tmp16_a[...] = jnp.where(iota16 == 0, mycnt, 0)
    pltpu.sync_copy(tmp16_a, sh_cnt.at[lp, q])
    plsc.subcore_barrier()

    @pl.when(q == 0)
    def _():
        def load_q(qq, _):
            pltpu.sync_copy(sh_cand.at[lp, qq], cand_v)
            pltpu.sync_copy(sh_cnt.at[lp, qq], tmp16_a)
            qcnt = tmp16_a[...][0]
            for j in range(QCAP // 16):
                vv = cand_v[pl.ds(j * 16, 16)]
                lanes = j * 16 + iota16
                call_v[pl.ds(qq * QCAP + j * 16, 16)] = jnp.where(lanes < qcnt, vv, PAD)
            return 0
        lax.fori_loop(0, 8, load_q, 0)

        def count_le(val):
            def cc(i, acc):
                v = call_v[pl.ds(i * 16, 16)]
                p = plsc.all_reduce_population_count(v <= val)
                p = p if getattr(p, "ndim", 0) == 0 else p[0]
                return acc + p
            return lax.fori_loop(0, 8 * QCAP // 16, cc, 0)

        def bisect(_, carry):
            lo, hi = carry
            mid = (lo + hi) >> 1
            ge = count_le(mid) >= kprime
            return (jnp.where(ge, lo, mid + 1), jnp.where(ge, mid, hi))
        lo, _hi = lax.fori_loop(0, BIN_SHIFT + POS_BITS, bisect,
                                (0, (1 << (BIN_SHIFT + POS_BITS)) - 1))

        outv = jnp.where(iota16 == 0, bsel, jnp.where(iota16 == 1, lo, 0))
        tmp16_a[...] = outv
        pltpu.sync_copy(tmp16_a, out_hbm.at[row])


def _select_thresholds(u):
    mesh = plsc.VectorSubcoreMesh(core_axis_name="c", subcore_axis_name="s")
    f = functools.partial(
        pl.kernel,
        mesh=mesh,
        out_type=jax.ShapeDtypeStruct((HROWS, 16), jnp.int32),
        compiler_params=pltpu.CompilerParams(needs_layout_passes=False,
                                             use_tc_tiling_on_sc=False),
        scratch_types=[
            pltpu.VMEM((HBINS,), jnp.int32),
            pltpu.VMEM((2, CH_HW, CD), jnp.int32),
            pltpu.VMEM((8, HBINS), jnp.int32),
            pltpu.VMEM((QCAP,), jnp.int32),
            pltpu.VMEM((8 * QCAP,), jnp.int32),
            pltpu.VMEM((16,), jnp.int32),
            pltpu.VMEM((16,), jnp.int32),
            pltpu.SemaphoreType.DMA((2,)),
            pltpu.VMEM_SHARED((2, 8, HBINS), jnp.int32),
            pltpu.VMEM_SHARED((2, 8, QCAP), jnp.int32),
            pltpu.VMEM_SHARED((2, 8, 16), jnp.int32),
            pltpu.VMEM_SHARED((2, 16), jnp.int32),
        ],
    )(_select_body)
    return f(u)


def _mask_body(x_ref, u_ref, chm_ref, thr_ref, o_ref):
    r = pl.program_id(0)
    k = pl.program_id(1)
    bsel = thr_ref[r, 0]
    kbound = thr_ref[r, 1]
    u = u_ref[0]
    top = u >> BIN_SHIFT
    i0 = lax.broadcasted_iota(jnp.int32, (HWB, CD), 0)
    i1 = lax.broadcasted_iota(jnp.int32, (HWB, CD), 1)
    pos = i1 * HW + k * HWB + i0
    keyv = ((u & LOW_MASK) << POS_BITS) | pos
    keep = (top < bsel) | ((top == bsel) & (keyv <= kbound))
    o_ref[0] = jnp.where(keep, x_ref[0] * chm_ref[...], 0.0)


def _apply_mask_a(xt, u_a, chm, thr_a):
    return pl.pallas_call(
        _mask_body,
        out_shape=jax.ShapeDtypeStruct((NROWS, HW, CD), jnp.float32),
        grid=(HROWS, ROW_CHUNKS),
        in_specs=[
            pl.BlockSpec((1, HWB, CD), lambda r, k: (r, k, 0)),
            pl.BlockSpec((1, HWB, CD), lambda r, k: (r, k, 0)),
            pl.BlockSpec((1, CD), lambda r, k: (0, 0)),
            pl.BlockSpec(memory_space=pltpu.SMEM),
        ],
        out_specs=pl.BlockSpec((1, HWB, CD), lambda r, k: (r, k, 0)),
    )(xt, u_a, chm, thr_a)


def _mask_body_b(buf_ref, x_ref, u_ref, chm_ref, thr_ref, o_ref):
    del buf_ref
    _mask_body(x_ref, u_ref, chm_ref, thr_ref, o_ref)


def _apply_mask_b(buf, xt, u_b, chm, thr_b):
    return pl.pallas_call(
        _mask_body_b,
        out_shape=jax.ShapeDtypeStruct((NROWS, HW, CD), jnp.float32),
        grid=(HROWS, ROW_CHUNKS),
        in_specs=[
            pl.BlockSpec(memory_space=pl.ANY),
            pl.BlockSpec((1, HWB, CD), lambda r, k: (r + HROWS, k, 0)),
            pl.BlockSpec((1, HWB, CD), lambda r, k: (r, k, 0)),
            pl.BlockSpec((1, CD), lambda r, k: (0, 0)),
            pl.BlockSpec(memory_space=pltpu.SMEM),
        ],
        out_specs=pl.BlockSpec((1, HWB, CD), lambda r, k: (r + HROWS, k, 0)),
        input_output_aliases={0: 0},
    )(buf, xt, u_b, chm, thr_b)


def kernel(x):
    xt = x.transpose(0, 2, 3, 1).reshape(NROWS, HW, CD)
    chm = jnp.asarray(_CHM)
    u_a = _gen_u(0)
    thr_a = _select_thresholds(u_a)
    u_b = _gen_u(HROWS)
    thr_b = _select_thresholds(u_b)
    out = _apply_mask_a(xt, u_a, chm, thr_a)
    out = _apply_mask_b(out, xt, u_b, chm, thr_b)
    return out.reshape(NROWS, HH, WW, CD).transpose(0, 3, 1, 2)
